# baseline (speedup 1.0000x reference)
import functools

import jax
import jax.numpy as jnp
from jax import lax
from jax.experimental import pallas as pl
from jax.experimental.pallas import tpu as pltpu
from jax.experimental.pallas import tpu_sc as plsc

B = 16384
L = 20
D = 64
V = 1000000
NC = 2
NS = 16
LANES = 16
NW = NC * NS
BPW = B // NW
CB = 32
NCHUNK = BPW // CB
GB = 4
NGRP = CB // GB
QPG = GB * L
NRED = QPG // LANES
NPAIR = CB * L
RB = 160
NB = NPAIR // RB


def _sc_body(center_hbm, context_hbm, emb_in_hbm, emb_out_hbm, out_hbm,
             cidx_v, xidx_v, in_rows_v, ctx_rows_v, tmp_v,
             out_v, sem_in, sem_ctx):
    wid = lax.axis_index("s") * NC + lax.axis_index("c")
    lane = lax.iota(jnp.int32, LANES)
    lane16 = lane * LANES
    zero16 = jnp.zeros((LANES,), jnp.int32)

    def _scalar(vec, t):
        return lax.reduce_max(
            lax.select(lane == t, vec, zero16), axes=(0,))

    def chunk_body(ci, carry):
        base_b = wid * BPW + ci * CB
        pltpu.sync_copy(center_hbm.at[pl.ds(base_b, CB)], cidx_v)
        pltpu.sync_copy(context_hbm.at[pl.ds(base_b * L, NPAIR)], xidx_v)

        def cin_body(j, c2):
            cv = cidx_v[pl.ds(j * LANES, LANES)]
            for t in range(LANES):
                pltpu.async_copy(emb_in_hbm.at[cv[t]],
                                 in_rows_v.at[j * LANES + t], sem_in)
            return c2

        lax.fori_loop(0, CB // LANES, cin_body, 0, unroll=False)

        def _issue(g):
            for h in range(RB // LANES):
                xv = xidx_v[pl.ds(g * RB + h * LANES, LANES)]
                for t in range(LANES):
                    pltpu.async_copy(
                        emb_out_hbm.at[xv[t]],
                        ctx_rows_v.at[g * RB + h * LANES + t], sem_ctx)

        _issue(0)

        def group_body(g4, carry2):
            b0 = g4 * GB
            a = [[in_rows_v[b0 + bb, pl.ds(c * LANES, LANES)]
                  for c in range(D // LANES)] for bb in range(GB)]
            for bb in range(GB):
                for l in range(L):
                    r = (b0 + bb) * L + l
                    q = a[bb][0] * ctx_rows_v[r, pl.ds(0, LANES)]
                    for c in range(1, D // LANES):
                        q = q + a[bb][c] * ctx_rows_v[r, pl.ds(c * LANES, LANES)]
                    tmp_v[pl.ds((bb * L + l) * LANES, LANES)] = q
            for g in range(NRED):
                acc = plsc.load_gather(tmp_v, [lane16 + g * (LANES * LANES)])
                for dd in range(1, LANES):
                    acc = acc + plsc.load_gather(
                        tmp_v, [lane16 + (g * (LANES * LANES) + dd)])
                out_v[pl.ds(g4 * QPG + g * LANES, LANES)] = acc
            return carry2

        pltpu.make_async_copy(
            emb_in_hbm.at[pl.ds(0, CB)], in_rows_v, sem_in).wait()

        def ctx_body(g, c2):
            _issue(g)
            pltpu.make_async_copy(
                emb_out_hbm.at[pl.ds(0, RB)],
                ctx_rows_v.at[pl.ds((g - 1) * RB, RB)],
                sem_ctx).wait()
            group_body(2 * (g - 1), 0)
            group_body(2 * (g - 1) + 1, 0)
            return c2

        lax.fori_loop(1, NB, ctx_body, 0, unroll=False)
        pltpu.make_async_copy(
            emb_out_hbm.at[pl.ds(0, RB)],
            ctx_rows_v.at[pl.ds((NB - 1) * RB, RB)],
            sem_ctx).wait()
        group_body(2 * (NB - 1), 0)
        group_body(2 * (NB - 1) + 1, 0)
        pltpu.sync_copy(out_v, out_hbm.at[pl.ds(base_b * L, NPAIR)])
        return carry

    lax.fori_loop(0, NCHUNK, chunk_body, 0, unroll=False)


@functools.partial(jax.jit, static_argnames=())
def _run(center_flat, context_flat, emb_in, emb_out):
    mesh = plsc.VectorSubcoreMesh(
        core_axis_name="c", subcore_axis_name="s",
        num_cores=NC, num_subcores=NS)
    grid_kernel = pl.kernel(
        _sc_body,
        out_type=jax.ShapeDtypeStruct((B * L,), jnp.float32),
        mesh=mesh,
        scratch_types=[
            pltpu.VMEM((CB,), jnp.int32),
            pltpu.VMEM((NPAIR,), jnp.int32),
            pltpu.VMEM((CB, D), jnp.float32),
            pltpu.VMEM((NPAIR, D), jnp.float32),
            pltpu.VMEM((QPG * LANES,), jnp.float32),
            pltpu.VMEM((NPAIR,), jnp.float32),
            pltpu.SemaphoreType.DMA,
            pltpu.SemaphoreType.DMA,
        ],
        compiler_params=pltpu.CompilerParams(
            needs_layout_passes=False, use_tc_tiling_on_sc=True),
    )
    return grid_kernel(center_flat, context_flat, emb_in, emb_out)


def kernel(center, context, emb_in, emb_out):
    center_flat = center.reshape(B)
    context_flat = context.reshape(B * L)
    out_flat = _run(center_flat, context_flat, emb_in, emb_out)
    return out_flat.reshape(B, L)


# speedup vs baseline: 1.1269x; 1.1269x over previous
import functools

import jax
import jax.numpy as jnp
from jax import lax
from jax.experimental import pallas as pl
from jax.experimental.pallas import tpu as pltpu
from jax.experimental.pallas import tpu_sc as plsc

B = 16384
L = 20
D = 64
V = 1000000
NC = 2
NS = 16
LANES = 16
NW = NC * NS
BPW = B // NW
CB = 32
NCHUNK = BPW // CB
GB = 4
NGRP = CB // GB
QPG = GB * L
NRED = QPG // LANES
NPAIR = CB * L
RB = 320
NB = NPAIR // RB


def _sc_body(center_hbm, context_hbm, emb_in_hbm, emb_out_hbm, out_hbm,
             cidx_v, xidx_v, in_rows_v, ctx_rows_v, tmp_v,
             out_v, sem_in, sem_ctx):
    wid = lax.axis_index("s") * NC + lax.axis_index("c")
    lane = lax.iota(jnp.int32, LANES)
    lane16 = lane * LANES
    zero16 = jnp.zeros((LANES,), jnp.int32)

    def _scalar(vec, t):
        return lax.reduce_max(
            lax.select(lane == t, vec, zero16), axes=(0,))

    def chunk_body(ci, carry):
        base_b = wid * BPW + ci * CB
        pltpu.sync_copy(center_hbm.at[pl.ds(base_b, CB)], cidx_v)
        pltpu.sync_copy(context_hbm.at[pl.ds(base_b * L, NPAIR)], xidx_v)

        def cin_body(j, c2):
            cv = cidx_v[pl.ds(j * LANES, LANES)]
            for t in range(LANES):
                pltpu.async_copy(emb_in_hbm.at[cv[t]],
                                 in_rows_v.at[j * LANES + t], sem_in)
            return c2

        lax.fori_loop(0, CB // LANES, cin_body, 0, unroll=False)

        def _issue(g):
            for h in range(RB // LANES):
                xv = xidx_v[pl.ds(g * RB + h * LANES, LANES)]
                for t in range(LANES):
                    pltpu.async_copy(
                        emb_out_hbm.at[xv[t]],
                        ctx_rows_v.at[g * RB + h * LANES + t], sem_ctx)

        _issue(0)

        def ctx_body(g, c2):
            _issue(g)
            pltpu.make_async_copy(
                emb_out_hbm.at[pl.ds(0, RB)],
                ctx_rows_v.at[pl.ds((g - 1) * RB, RB)],
                sem_ctx).wait()
            return c2

        lax.fori_loop(1, NB, ctx_body, 0, unroll=False)
        pltpu.make_async_copy(
            emb_out_hbm.at[pl.ds(0, RB)],
            ctx_rows_v.at[pl.ds((NB - 1) * RB, RB)],
            sem_ctx).wait()
        pltpu.make_async_copy(
            emb_in_hbm.at[pl.ds(0, CB)], in_rows_v, sem_in).wait()

        def group_body(g4, carry2):
            b0 = g4 * GB
            a = [[in_rows_v[b0 + bb, pl.ds(c * LANES, LANES)]
                  for c in range(D // LANES)] for bb in range(GB)]
            for bb in range(GB):
                for l in range(L):
                    r = (b0 + bb) * L + l
                    q = a[bb][0] * ctx_rows_v[r, pl.ds(0, LANES)]
                    for c in range(1, D // LANES):
                        q = q + a[bb][c] * ctx_rows_v[r, pl.ds(c * LANES, LANES)]
                    tmp_v[pl.ds((bb * L + l) * LANES, LANES)] = q
            for g in range(NRED):
                acc = plsc.load_gather(tmp_v, [lane16 + g * (LANES * LANES)])
                for dd in range(1, LANES):
                    acc = acc + plsc.load_gather(
                        tmp_v, [lane16 + (g * (LANES * LANES) + dd)])
                out_v[pl.ds(g4 * QPG + g * LANES, LANES)] = acc
            return carry2

        lax.fori_loop(0, NGRP, group_body, 0, unroll=False)
        pltpu.sync_copy(out_v, out_hbm.at[pl.ds(base_b * L, NPAIR)])
        return carry

    lax.fori_loop(0, NCHUNK, chunk_body, 0, unroll=False)


@functools.partial(jax.jit, static_argnames=())
def _run(center_flat, context_flat, emb_in, emb_out):
    mesh = plsc.VectorSubcoreMesh(
        core_axis_name="c", subcore_axis_name="s",
        num_cores=NC, num_subcores=NS)
    grid_kernel = pl.kernel(
        _sc_body,
        out_type=jax.ShapeDtypeStruct((B * L,), jnp.float32),
        mesh=mesh,
        scratch_types=[
            pltpu.VMEM((CB,), jnp.int32),
            pltpu.VMEM((NPAIR,), jnp.int32),
            pltpu.VMEM((CB, D), jnp.float32),
            pltpu.VMEM((NPAIR, D), jnp.float32),
            pltpu.VMEM((QPG * LANES,), jnp.float32),
            pltpu.VMEM((NPAIR,), jnp.float32),
            pltpu.SemaphoreType.DMA,
            pltpu.SemaphoreType.DMA,
        ],
        compiler_params=pltpu.CompilerParams(
            needs_layout_passes=False, use_tc_tiling_on_sc=True),
    )
    return grid_kernel(center_flat, context_flat, emb_in, emb_out)


def kernel(center, context, emb_in, emb_out):
    center_flat = center.reshape(B)
    context_flat = context.reshape(B * L)
    out_flat = _run(center_flat, context_flat, emb_in, emb_out)
    return out_flat.reshape(B, L)


# speedup vs baseline: 1.1387x; 1.0105x over previous
import functools

import jax
import jax.numpy as jnp
from jax import lax
from jax.experimental import pallas as pl
from jax.experimental.pallas import tpu as pltpu
from jax.experimental.pallas import tpu_sc as plsc

B = 16384
L = 20
D = 64
V = 1000000
NC = 2
NS = 16
LANES = 16
NW = NC * NS
BPW = B // NW
CB = 32
NCHUNK = BPW // CB
GB = 4
NGRP = CB // GB
QPG = GB * L
NRED = QPG // LANES
NPAIR = CB * L
RB = 128
NB = NPAIR // RB


def _sc_body(center_hbm, context_hbm, emb_in_hbm, emb_out_hbm, out_hbm,
             cidx_v, xidx_v, in_rows_v, ctx_rows_v, tmp_v,
             out_v, sem_in, sem_ctx):
    wid = lax.axis_index("s") * NC + lax.axis_index("c")
    lane = lax.iota(jnp.int32, LANES)
    lane16 = lane * LANES
    zero16 = jnp.zeros((LANES,), jnp.int32)

    def _scalar(vec, t):
        return lax.reduce_max(
            lax.select(lane == t, vec, zero16), axes=(0,))

    def chunk_body(ci, carry):
        base_b = wid * BPW + ci * CB
        pltpu.sync_copy(center_hbm.at[pl.ds(base_b, CB)], cidx_v)
        pltpu.sync_copy(context_hbm.at[pl.ds(base_b * L, NPAIR)], xidx_v)

        def cin_body(j, c2):
            cv = cidx_v[pl.ds(j * LANES, LANES)]
            for t in range(LANES):
                pltpu.async_copy(emb_in_hbm.at[cv[t]],
                                 in_rows_v.at[j * LANES + t], sem_in)
            return c2

        lax.fori_loop(0, CB // LANES, cin_body, 0, unroll=False)

        def _issue(g):
            for h in range(RB // LANES):
                xv = xidx_v[pl.ds(g * RB + h * LANES, LANES)]
                for t in range(LANES):
                    pltpu.async_copy(
                        emb_out_hbm.at[xv[t]],
                        ctx_rows_v.at[g * RB + h * LANES + t], sem_ctx)

        _issue(0)

        def ctx_body(g, c2):
            _issue(g)
            pltpu.make_async_copy(
                emb_out_hbm.at[pl.ds(0, RB)],
                ctx_rows_v.at[pl.ds((g - 1) * RB, RB)],
                sem_ctx).wait()
            return c2

        lax.fori_loop(1, NB, ctx_body, 0, unroll=False)
        pltpu.make_async_copy(
            emb_out_hbm.at[pl.ds(0, RB)],
            ctx_rows_v.at[pl.ds((NB - 1) * RB, RB)],
            sem_ctx).wait()
        pltpu.make_async_copy(
            emb_in_hbm.at[pl.ds(0, CB)], in_rows_v, sem_in).wait()

        def group_body(g4, carry2):
            b0 = g4 * GB
            a = [[in_rows_v[b0 + bb, pl.ds(c * LANES, LANES)]
                  for c in range(D // LANES)] for bb in range(GB)]
            for bb in range(GB):
                for l in range(L):
                    r = (b0 + bb) * L + l
                    q = a[bb][0] * ctx_rows_v[r, pl.ds(0, LANES)]
                    for c in range(1, D // LANES):
                        q = q + a[bb][c] * ctx_rows_v[r, pl.ds(c * LANES, LANES)]
                    tmp_v[pl.ds((bb * L + l) * LANES, LANES)] = q
            for g in range(NRED):
                acc = plsc.load_gather(tmp_v, [lane16 + g * (LANES * LANES)])
                for dd in range(1, LANES):
                    acc = acc + plsc.load_gather(
                        tmp_v, [lane16 + (g * (LANES * LANES) + dd)])
                out_v[pl.ds(g4 * QPG + g * LANES, LANES)] = acc
            return carry2

        lax.fori_loop(0, NGRP, group_body, 0, unroll=False)
        pltpu.sync_copy(out_v, out_hbm.at[pl.ds(base_b * L, NPAIR)])
        return carry

    lax.fori_loop(0, NCHUNK, chunk_body, 0, unroll=False)


@functools.partial(jax.jit, static_argnames=())
def _run(center_flat, context_flat, emb_in, emb_out):
    mesh = plsc.VectorSubcoreMesh(
        core_axis_name="c", subcore_axis_name="s",
        num_cores=NC, num_subcores=NS)
    grid_kernel = pl.kernel(
        _sc_body,
        out_type=jax.ShapeDtypeStruct((B * L,), jnp.float32),
        mesh=mesh,
        scratch_types=[
            pltpu.VMEM((CB,), jnp.int32),
            pltpu.VMEM((NPAIR,), jnp.int32),
            pltpu.VMEM((CB, D), jnp.float32),
            pltpu.VMEM((NPAIR, D), jnp.float32),
            pltpu.VMEM((QPG * LANES,), jnp.float32),
            pltpu.VMEM((NPAIR,), jnp.float32),
            pltpu.SemaphoreType.DMA,
            pltpu.SemaphoreType.DMA,
        ],
        compiler_params=pltpu.CompilerParams(
            needs_layout_passes=False, use_tc_tiling_on_sc=True),
    )
    return grid_kernel(center_flat, context_flat, emb_in, emb_out)


def kernel(center, context, emb_in, emb_out):
    center_flat = center.reshape(B)
    context_flat = context.reshape(B * L)
    out_flat = _run(center_flat, context_flat, emb_in, emb_out)
    return out_flat.reshape(B, L)
